# baseline (speedup 1.0000x reference)
import functools

import jax
import jax.numpy as jnp
from jax import lax
from jax.experimental import pallas as pl
from jax.experimental.pallas import tpu as pltpu
from jax.experimental.pallas import tpu_sc as plsc

_NUM_CORES = 2
_NUM_SUBCORES = 16
_NUM_WORKERS = _NUM_CORES * _NUM_SUBCORES
_LANES = 16


def _build(V, D, B):
    n_groups = _NUM_WORKERS // 2
    b_per_g = B // n_groups
    half = D // 2
    chunk = _LANES
    n_chunks = b_per_g // chunk
    mesh = plsc.VectorSubcoreMesh(core_axis_name="c", subcore_axis_name="s")

    @functools.partial(
        pl.kernel,
        mesh=mesh,
        out_type=jax.ShapeDtypeStruct((D, B), jnp.float32),
        scratch_types=[
            pltpu.VMEM((b_per_g,), jnp.int32),
            pltpu.VMEM((3, chunk, half, 128), jnp.float32),
            pltpu.VMEM((half, b_per_g), jnp.float32),
            pltpu.SemaphoreType.DMA,
        ],
        compiler_params=pltpu.CompilerParams(needs_layout_passes=False),
    )
    def gather_kernel(tableT_hbm, idx_hbm, outT_hbm, idx_v, panel_v, colsT_v,
                      sem):
        wid = lax.axis_index("s") * _NUM_CORES + lax.axis_index("c")
        h = lax.rem(wid, 2)
        grp = lax.div(wid, 2)
        row0 = pl.multiple_of(h * half, half)
        base = grp * b_per_g
        pltpu.sync_copy(idx_hbm.at[pl.ds(base, b_per_g)], idx_v)

        slot = lax.iota(jnp.int32, _LANES)

        def _fire(c, b):
            iv = idx_v[pl.ds(c * chunk, chunk)]
            for l in range(chunk):
                col0 = pl.multiple_of(
                    lax.shift_left(lax.shift_right_logical(iv[l], 7), 7), 128
                )
                pltpu.async_copy(
                    tableT_hbm.at[pl.ds(row0, half), pl.ds(col0, 128)],
                    panel_v.at[b, l],
                    sem,
                )

        def _drain_extract(c, b):
            for l in range(chunk):
                pltpu.make_async_copy(
                    tableT_hbm.at[pl.ds(0, half), pl.ds(0, 128)],
                    panel_v.at[b, l],
                    sem,
                ).wait()
            iv = idx_v[pl.ds(c * chunk, chunk)]
            lanes = jnp.bitwise_and(iv, 127)
            for q in range(half):
                qv = jnp.full((_LANES,), q, jnp.int32)
                vals = plsc.load_gather(panel_v.at[b], [slot, qv, lanes])
                colsT_v[q, pl.ds(c * chunk, chunk)] = vals

        _fire(0, 0)
        _fire(1, 1)
        _fire(2, 2)

        def _step(go, _):
            for b in range(3):
                c = go * 3 + b
                _drain_extract(c - 3, b)
                _fire(c, b)
            return _

        lax.fori_loop(1, (n_chunks - 1) // 3, _step, 0)
        _drain_extract(n_chunks - 4, 0)
        _fire(n_chunks - 1, 0)
        _drain_extract(n_chunks - 3, 1)
        _drain_extract(n_chunks - 2, 2)
        _drain_extract(n_chunks - 1, 0)

        out_base = pl.multiple_of(base, 128)
        pltpu.sync_copy(
            colsT_v,
            outT_hbm.at[pl.ds(row0, half), pl.ds(out_base, b_per_g)],
        )

    return gather_kernel


_GATHER = _build(1000000, 32, 16384)


@jax.jit
def kernel(table, indices):
    outT = _GATHER(table.T, indices.astype(jnp.int32))
    return outT.T


# speedup vs baseline: 1.0306x; 1.0306x over previous
import functools

import jax
import jax.numpy as jnp
from jax import lax
from jax.experimental import pallas as pl
from jax.experimental.pallas import tpu as pltpu
from jax.experimental.pallas import tpu_sc as plsc

_NUM_CORES = 2
_NUM_SUBCORES = 16
_NUM_WORKERS = _NUM_CORES * _NUM_SUBCORES
_LANES = 16


def _build(V, D, B):
    n_groups = _NUM_WORKERS // 2
    b_per_g = B // n_groups
    half = D // 2
    chunk = _LANES
    n_chunks = b_per_g // chunk
    mesh = plsc.VectorSubcoreMesh(core_axis_name="c", subcore_axis_name="s")

    @functools.partial(
        pl.kernel,
        mesh=mesh,
        out_type=jax.ShapeDtypeStruct((D, B), jnp.float32),
        scratch_types=[
            pltpu.VMEM((b_per_g,), jnp.int32),
            pltpu.VMEM((2, chunk, half, 128), jnp.float32),
            pltpu.VMEM((half, b_per_g), jnp.float32),
            pltpu.SemaphoreType.DMA,
        ],
        compiler_params=pltpu.CompilerParams(needs_layout_passes=False),
    )
    def gather_kernel(tableT_hbm, idx_hbm, outT_hbm, idx_v, panel_v, colsT_v,
                      sem):
        wid = lax.axis_index("s") * _NUM_CORES + lax.axis_index("c")
        h = lax.rem(wid, 2)
        grp = lax.div(wid, 2)
        row0 = pl.multiple_of(h * half, half)
        base = grp * b_per_g
        pltpu.sync_copy(idx_hbm.at[pl.ds(base, b_per_g)], idx_v)

        slot = lax.iota(jnp.int32, _LANES)

        def _fire(c, b):
            iv = idx_v[pl.ds(c * chunk, chunk)]
            for l in range(chunk):
                col0 = pl.multiple_of(
                    lax.shift_left(lax.shift_right_logical(iv[l], 7), 7), 128
                )
                pltpu.async_copy(
                    tableT_hbm.at[pl.ds(row0, half), pl.ds(col0, 128)],
                    panel_v.at[b, l],
                    sem,
                )

        def _drain_extract(c, b):
            for l in range(chunk):
                pltpu.make_async_copy(
                    tableT_hbm.at[pl.ds(0, half), pl.ds(0, 128)],
                    panel_v.at[b, l],
                    sem,
                ).wait()
            iv = idx_v[pl.ds(c * chunk, chunk)]
            lanes = jnp.bitwise_and(iv, 127)
            for q in range(half):
                qv = jnp.full((_LANES,), q, jnp.int32)
                vals = plsc.load_gather(panel_v.at[b], [slot, qv, lanes])
                colsT_v[q, pl.ds(c * chunk, chunk)] = vals

        _fire(0, 0)
        _fire(1, 1)

        def _step(go, _):
            for b in range(2):
                c = go * 2 + b
                _drain_extract(c - 2, b)
                _fire(c, b)
            return _

        lax.fori_loop(1, n_chunks // 2, _step, 0)
        _drain_extract(n_chunks - 2, 0)
        _drain_extract(n_chunks - 1, 1)

        out_base = pl.multiple_of(base, 128)
        pltpu.sync_copy(
            colsT_v,
            outT_hbm.at[pl.ds(row0, half), pl.ds(out_base, b_per_g)],
        )

    return gather_kernel


_GATHER = _build(1000000, 32, 16384)


@jax.jit
def kernel(table, indices):
    outT = _GATHER(table.T, indices.astype(jnp.int32))
    return outT.T
